# baseline (speedup 1.0000x reference)
import functools

import jax
import jax.numpy as jnp
from jax.experimental import pallas as pl
from jax.experimental.pallas import tpu as pltpu

N = 10000
D = 128
C = 16
K = 5000
R = 10
M = N // R


NCH = 4
CHR = N // NCH


def _in_cp(x_hbm, xb_ref, sems, j):
    return pltpu.make_async_copy(
        x_hbm.at[pl.ds(j * CHR, CHR), :], xb_ref.at[j % 2], sems.at[j % 2])


def _body(x_hbm, w_ref, b_ref, wp_ref, xf_ref, predt_ref, zr_ref, zs_ref,
          ws_ref, xb_ref, sems):
    w = w_ref[:, :]
    bt = b_ref[:, :]
    wp = wp_ref[:, :]

    _in_cp(x_hbm, xb_ref, sems, 0).start()
    for j in range(NCH):
        if j + 1 < NCH:
            _in_cp(x_hbm, xb_ref, sems, j + 1).start()
        _in_cp(x_hbm, xb_ref, sems, j).wait()
        pt = jax.lax.dot_general(
            w, xb_ref[j % 2], (((1,), (1,)), ((), ())),
            preferred_element_type=jnp.float32) + bt
        predt_ref[:, pl.ds(j * CHR, CHR)] = pt
        zj = jax.lax.dot_general(
            wp, pt, (((1,), (0,)), ((), ())),
            preferred_element_type=jnp.float32)
        zr_ref[0:1, pl.ds(j * CHR, CHR)] = zj

    predt = predt_ref[:, :]

    for j in range(R):
        zs_ref[j:j + 1, :] = zr_ref[0:1, pl.ds(j * M, M)]
    zd = zs_ref[:, :]

    u = jax.lax.bitcast_convert_type(zd, jnp.uint32)
    sign = u >> jnp.uint32(31)
    flip = jnp.where(sign == jnp.uint32(1),
                     jnp.uint32(0xFFFFFFFF), jnp.uint32(0x80000000))
    key = u ^ flip

    def _cnt_ge(t):
        return jnp.sum((key >= t).astype(jnp.int32))

    kth = jnp.uint32(0)
    for sh in range(28, -1, -4):
        cnts = [_cnt_ge(kth | jnp.uint32(d << sh)) for d in range(1, 16)]
        digit = sum((c >= K).astype(jnp.uint32) for c in cnts)
        kth = kth | (digit << jnp.uint32(sh))

    above = key > kth
    m = jnp.sum(above.astype(jnp.int32))
    need = K - m

    tie = key == kth
    idx = (jax.lax.broadcasted_iota(jnp.int32, (R, M), 0) * M
           + jax.lax.broadcasted_iota(jnp.int32, (R, M), 1))

    def _cnt_lt(v):
        return jnp.sum((tie & (idx < v)).astype(jnp.int32))

    def _tie_radix():
        js = jnp.int32(0)
        for sh in range(12, -1, -4):
            cnts = [_cnt_lt(js | jnp.int32(d << sh)) for d in range(1, 16)]
            digit = sum((c < need).astype(jnp.int32) for c in cnts)
            js = js | (digit << sh)
        return js

    t_total = jnp.sum(tie.astype(jnp.int32))
    jstar = jax.lax.cond(need == t_total, lambda: jnp.int32(N - 1),
                         _tie_radix)

    sel = above | (tie & (idx <= jstar))
    norm = jnp.sqrt(jnp.sum(wp * wp)) + 1e-16
    wgt = jnp.where(sel, jnp.tanh(zd / norm), 0.0)

    for j in range(R):
        ws_ref[0:1, pl.ds(j * M, M)] = wgt[j:j + 1, :]

    acc = jax.lax.dot_general(
        ws_ref[:, :], predt, (((1,), (1,)), ((), ())),
        preferred_element_type=jnp.float32)
    xf_ref[:, :] = acc * (1.0 / K)


@functools.partial(jax.jit, static_argnames=())
def kernel(x, edge_index, batch, W, b, w_pool):
    del edge_index, batch
    bt = b.reshape(C, 1)
    wp2 = w_pool.reshape(1, C)
    x_final, predt = pl.pallas_call(
        _body,
        in_specs=[
            pl.BlockSpec(memory_space=pl.ANY),
            pl.BlockSpec(memory_space=pltpu.MemorySpace.VMEM),
            pl.BlockSpec(memory_space=pltpu.MemorySpace.VMEM),
            pl.BlockSpec(memory_space=pltpu.MemorySpace.VMEM),
        ],
        out_shape=(
            jax.ShapeDtypeStruct((1, C), jnp.float32),
            jax.ShapeDtypeStruct((C, N), jnp.float32),
        ),
        scratch_shapes=[
            pltpu.VMEM((1, N), jnp.float32),
            pltpu.VMEM((R, M), jnp.float32),
            pltpu.VMEM((1, N), jnp.float32),
            pltpu.VMEM((2, CHR, D), jnp.float32),
            pltpu.SemaphoreType.DMA((2,)),
        ],
    )(x, W, bt, wp2)
    return (x_final, predt.T)


# speedup vs baseline: 1.1515x; 1.1515x over previous
import functools

import jax
import jax.numpy as jnp
from jax.experimental import pallas as pl
from jax.experimental.pallas import tpu as pltpu

N = 10000
D = 128
C = 16
K = 5000
R = 10
M = N // R


def _body(x_ref, w_ref, b_ref, wp_ref, xf_ref, predt_ref, zr_ref, zs_ref,
          ws_ref):
    x = x_ref[:, :]
    w = w_ref[:, :]
    bt = b_ref[:, :]
    wp = wp_ref[:, :]

    predt = jax.lax.dot_general(
        w, x, (((1,), (1,)), ((), ())), preferred_element_type=jnp.float32
    ) + bt
    predt_ref[:, :] = predt

    z = jax.lax.dot_general(
        wp, predt, (((1,), (0,)), ((), ())),
        preferred_element_type=jnp.float32)
    zr_ref[:, :] = z

    for j in range(R):
        zs_ref[j:j + 1, :] = zr_ref[0:1, pl.ds(j * M, M)]
    zd = zs_ref[:, :]

    u = jax.lax.bitcast_convert_type(zd, jnp.uint32)
    sign = u >> jnp.uint32(31)
    flip = jnp.where(sign == jnp.uint32(1),
                     jnp.uint32(0xFFFFFFFF), jnp.uint32(0x80000000))
    key = u ^ flip

    def _cnt_ge(t):
        return jnp.sum((key >= t).astype(jnp.int32))

    kth = jnp.uint32(0)
    for sh in range(28, -1, -4):
        cnts = [_cnt_ge(kth | jnp.uint32(d << sh)) for d in range(1, 16)]
        digit = sum((c >= K).astype(jnp.uint32) for c in cnts)
        kth = kth | (digit << jnp.uint32(sh))

    above = key > kth
    m = jnp.sum(above.astype(jnp.int32))
    need = K - m

    tie = key == kth
    idx = (jax.lax.broadcasted_iota(jnp.int32, (R, M), 0) * M
           + jax.lax.broadcasted_iota(jnp.int32, (R, M), 1))

    def _cnt_lt(v):
        return jnp.sum((tie & (idx < v)).astype(jnp.int32))

    def _tie_radix():
        js = jnp.int32(0)
        for sh in range(12, -1, -4):
            cnts = [_cnt_lt(js | jnp.int32(d << sh)) for d in range(1, 16)]
            digit = sum((c < need).astype(jnp.int32) for c in cnts)
            js = js | (digit << sh)
        return js

    t_total = jnp.sum(tie.astype(jnp.int32))
    jstar = jax.lax.cond(need == t_total, lambda: jnp.int32(N - 1),
                         _tie_radix)

    sel = above | (tie & (idx <= jstar))
    norm = jnp.sqrt(jnp.sum(wp * wp)) + 1e-16
    wgt = jnp.where(sel, jnp.tanh(zd / norm), 0.0)

    for j in range(R):
        ws_ref[0:1, pl.ds(j * M, M)] = wgt[j:j + 1, :]

    acc = jax.lax.dot_general(
        ws_ref[:, :], predt, (((1,), (1,)), ((), ())),
        preferred_element_type=jnp.float32)
    xf_ref[:, :] = acc * (1.0 / K)


@functools.partial(jax.jit, static_argnames=())
def kernel(x, edge_index, batch, W, b, w_pool):
    del edge_index, batch
    bt = b.reshape(C, 1)
    wp2 = w_pool.reshape(1, C)
    x_final, predt = pl.pallas_call(
        _body,
        out_shape=(
            jax.ShapeDtypeStruct((1, C), jnp.float32),
            jax.ShapeDtypeStruct((C, N), jnp.float32),
        ),
        scratch_shapes=[
            pltpu.VMEM((1, N), jnp.float32),
            pltpu.VMEM((R, M), jnp.float32),
            pltpu.VMEM((1, N), jnp.float32),
        ],
    )(x, W, bt, wp2)
    return (x_final, predt.T)
